# baseline (speedup 1.0000x reference)
import functools

import jax
import jax.numpy as jnp
from jax import lax
from jax.experimental import pallas as pl
from jax.experimental.pallas import tpu as pltpu
from jax.experimental.pallas import tpu_sc as plsc

E = 8
TOPK = 2
D = 2048
H = 4096
O = 2048
N = 2048

BM = 512
BD = 512
BH2 = 512
ND = D // BD
NH2 = H // BH2
NJ = ND + NH2
T = N * TOPK
TPAD = T + E * BM
NT = TPAD // BM

BR = 256

NC = 2
NS = 16
NW = NC * NS

GCH = 16
CCH = 16


def _router_body(x_ref, wr_ref, br_ref, vals_ref, idx_ref):
    logits = jnp.dot(x_ref[...], wr_ref[...],
                     preferred_element_type=jnp.float32) + br_ref[...]
    m = jnp.max(logits, axis=1, keepdims=True)
    ex = jnp.exp(logits - m)
    probs = ex / jnp.sum(ex, axis=1, keepdims=True)
    iota = lax.broadcasted_iota(jnp.int32, probs.shape, 1)
    m1 = jnp.max(probs, axis=1, keepdims=True)
    i1 = jnp.min(jnp.where(probs == m1, iota, E), axis=1, keepdims=True)
    probs2 = jnp.where(iota == i1, -1.0, probs)
    m2 = jnp.max(probs2, axis=1, keepdims=True)
    i2 = jnp.min(jnp.where(probs2 == m2, iota, E), axis=1, keepdims=True)
    vals_ref[...] = jnp.concatenate([m1, m2], axis=1)
    idx_ref[...] = jnp.concatenate([i1, i2], axis=1)


def _router(x, Wr, br2):
    return pl.pallas_call(
        _router_body,
        grid=(N // BR,),
        in_specs=[
            pl.BlockSpec((BR, D), lambda i: (i, 0)),
            pl.BlockSpec((D, E), lambda i: (0, 0)),
            pl.BlockSpec((1, E), lambda i: (0, 0)),
        ],
        out_specs=[
            pl.BlockSpec((BR, TOPK), lambda i: (i, 0)),
            pl.BlockSpec((BR, TOPK), lambda i: (i, 0)),
        ],
        out_shape=[
            jax.ShapeDtypeStruct((N, TOPK), jnp.float32),
            jax.ShapeDtypeStruct((N, TOPK), jnp.int32),
        ],
    )(x, Wr, br2)


@functools.cache
def _make_gather_x():
    mesh = plsc.VectorSubcoreMesh(core_axis_name="c", subcore_axis_name="s")

    rpw = TPAD // NW
    nch = rpw // GCH

    @functools.partial(
        pl.kernel,
        mesh=mesh,
        out_type=jax.ShapeDtypeStruct((TPAD, D), jnp.float32),
        scratch_types=[
            pltpu.VMEM((rpw,), jnp.int32),
            pltpu.VMEM((2, GCH, D), jnp.float32),
            pltpu.SemaphoreType.DMA,
            pltpu.SemaphoreType.DMA,
            pltpu.SemaphoreType.DMA,
            pltpu.SemaphoreType.DMA,
        ],
    )
    def _gather_x(x_hbm, idx_hbm, out_hbm, idxs_v, rows_v, gs0, gs1, os0, os1):
        wid = lax.axis_index("s") * NC + lax.axis_index("c")
        base = wid * rpw
        pltpu.sync_copy(idx_hbm.at[pl.ds(base, rpw)], idxs_v)
        gsems = (gs0, gs1)
        osems = (os0, os1)

        def gstart(i, buf):
            return pltpu.async_copy(
                x_hbm.at[idxs_v.at[pl.ds(i * GCH, GCH)]],
                rows_v.at[buf], gsems[buf])

        gcp = gstart(0, 0)
        ocp = [None, None]
        for i in range(nch):
            buf = i % 2
            nxt_cp = None
            if i + 1 < nch:
                nbuf = (i + 1) % 2
                if ocp[nbuf] is not None:
                    ocp[nbuf].wait()
                    ocp[nbuf] = None
                nxt_cp = gstart(i + 1, nbuf)
            gcp.wait()
            ocp[buf] = pltpu.async_copy(
                rows_v.at[buf], out_hbm.at[pl.ds(base + i * GCH, GCH)],
                osems[buf])
            gcp = nxt_cp
        for buf in range(2):
            if ocp[buf] is not None:
                ocp[buf].wait()

    return _gather_x


def _mlp_body(te_ref, tv_ref, xs_ref, w1_ref, b1_ref, w2_ref, b2_ref, g_ref,
              ys_ref, hb_ref, acc_ref):
    t = pl.program_id(0)
    j = pl.program_id(1)
    valid = tv_ref[t] == 1

    @pl.when(j == 0)
    def _():
        acc_ref[...] = jnp.zeros_like(acc_ref)

    @pl.when((j < ND) & valid)
    def _():
        part = jnp.dot(xs_ref[...].astype(jnp.bfloat16),
                       w1_ref[0].astype(jnp.bfloat16),
                       preferred_element_type=jnp.float32)

        @pl.when(j == 0)
        def _():
            hb_ref[...] = part

        @pl.when(j > 0)
        def _():
            hb_ref[...] += part

    @pl.when((j == ND - 1) & valid)
    def _():
        hb_ref[...] = jnp.maximum(hb_ref[...] + b1_ref[0], 0.0)

    @pl.when((j >= ND) & valid)
    def _():
        k = j - ND
        hblk = hb_ref[:, pl.ds(k * BH2, BH2)]
        acc_ref[...] += jnp.dot(hblk.astype(jnp.bfloat16),
                                w2_ref[0].astype(jnp.bfloat16),
                                preferred_element_type=jnp.float32)

    @pl.when(j == NJ - 1)
    def _():
        ys_ref[...] = (acc_ref[...] + b2_ref[0]) * g_ref[...]


def _mlp(te, tv, xs, W1, b1r, W2, b2r, gates_col):
    ndm1 = ND - 1
    nh2m1 = NH2 - 1
    grid_spec = pltpu.PrefetchScalarGridSpec(
        num_scalar_prefetch=2,
        grid=(NT, NJ),
        in_specs=[
            pl.BlockSpec((BM, BD),
                         lambda t, j, te, tv: (t, jnp.minimum(j, ndm1))),
            pl.BlockSpec((1, BD, H),
                         lambda t, j, te, tv: (te[t], jnp.minimum(j, ndm1), 0)),
            pl.BlockSpec((1, 1, H), lambda t, j, te, tv: (te[t], 0, 0)),
            pl.BlockSpec((1, BH2, O),
                         lambda t, j, te, tv:
                         (te[t], jnp.clip(j - ND, 0, nh2m1), 0)),
            pl.BlockSpec((1, 1, O), lambda t, j, te, tv: (te[t], 0, 0)),
            pl.BlockSpec((BM, 1), lambda t, j, te, tv: (t, 0)),
        ],
        out_specs=pl.BlockSpec((BM, O), lambda t, j, te, tv: (t, 0)),
        scratch_shapes=[
            pltpu.VMEM((BM, H), jnp.float32),
            pltpu.VMEM((BM, O), jnp.float32),
        ],
    )
    return pl.pallas_call(
        _mlp_body,
        grid_spec=grid_spec,
        out_shape=jax.ShapeDtypeStruct((TPAD, O), jnp.float32),
        compiler_params=pltpu.CompilerParams(
            dimension_semantics=("arbitrary", "arbitrary")),
    )(te, tv, xs, W1, b1r, W2, b2r, gates_col)


@functools.cache
def _make_combine():
    mesh = plsc.VectorSubcoreMesh(core_axis_name="c", subcore_axis_name="s")

    @functools.partial(
        pl.kernel,
        mesh=mesh,
        out_type=jax.ShapeDtypeStruct((N, O), jnp.float32),
        scratch_types=[
            pltpu.VMEM((CCH,), jnp.int32),
            pltpu.VMEM((CCH,), jnp.int32),
            pltpu.VMEM((CCH, O), jnp.float32),
            pltpu.VMEM((CCH, O), jnp.float32),
            pltpu.SemaphoreType.DMA,
        ],
    )
    def _combine(ys_hbm, d0_hbm, d1_hbm, out_hbm, i0_v, i1_v, r0_v, r1_v, sem):
        wid = lax.axis_index("s") * NC + lax.axis_index("c")
        rpw = N // NW
        base = wid * rpw

        def chunk(c, carry):
            off = base + c * CCH
            pltpu.sync_copy(d0_hbm.at[pl.ds(off, CCH)], i0_v)
            pltpu.sync_copy(d1_hbm.at[pl.ds(off, CCH)], i1_v)
            cp0 = pltpu.async_copy(ys_hbm.at[i0_v], r0_v, sem)
            cp1 = pltpu.async_copy(ys_hbm.at[i1_v], r1_v, sem)
            cp0.wait()
            cp1.wait()

            def add_lanes(j, carry2):
                s = pl.ds(j * 16, 16)
                for r in range(CCH):
                    r0_v[r, s] = r0_v[r, s] + r1_v[r, s]
                return carry2

            lax.fori_loop(0, O // 16, add_lanes, 0)
            pltpu.sync_copy(r0_v, out_hbm.at[pl.ds(off, CCH)])
            return carry

        lax.fori_loop(0, rpw // CCH, chunk, 0)

    return _combine


def kernel(x, Wr, br, W1, b1, W2, b2):
    top_vals, top_idx = _router(x, Wr, br.reshape(1, E))

    flat_e = top_idx.reshape(-1)
    flat_gate = top_vals.reshape(-1)
    flat_tok = jnp.repeat(jnp.arange(N, dtype=jnp.int32), TOPK)
    onehot = (flat_e[:, None] == jnp.arange(E, dtype=jnp.int32)[None, :])
    csum = jnp.cumsum(onehot.astype(jnp.int32), axis=0)
    counts = csum[-1]
    rank = jnp.take_along_axis(csum, flat_e[:, None], axis=1)[:, 0] - 1
    padded = ((counts + BM - 1) // BM) * BM
    seg_end = jnp.cumsum(padded)
    seg_start = seg_end - padded
    total_active = seg_end[-1]
    dest = seg_start[flat_e] + rank

    spread = jnp.arange(TPAD, dtype=jnp.int32) % N
    tok_pad = spread.at[dest].set(flat_tok)
    gates_pad = jnp.zeros((TPAD,), jnp.float32).at[dest].set(flat_gate)

    tile_start = jnp.arange(NT, dtype=jnp.int32) * BM
    te_raw = jnp.searchsorted(seg_end, tile_start, side="right").astype(jnp.int32)
    valid = tile_start < total_active
    num_active = total_active // BM
    last_e = te_raw[jnp.maximum(num_active - 1, 0)]
    te = jnp.where(valid, jnp.minimum(te_raw, E - 1), last_e)
    tv = valid.astype(jnp.int32)

    xs = _make_gather_x()(x, tok_pad)
    ys = _mlp(te, tv, xs, W1, b1.reshape(E, 1, H), W2, b2.reshape(E, 1, O),
              gates_pad.reshape(TPAD, 1))

    d0 = dest[0::2].astype(jnp.int32)
    d1 = dest[1::2].astype(jnp.int32)
    return _make_combine()(ys, d0, d1)


# speedup vs baseline: 1.2413x; 1.2413x over previous
import functools

import jax
import jax.numpy as jnp
from jax import lax
from jax.experimental import pallas as pl
from jax.experimental.pallas import tpu as pltpu
from jax.experimental.pallas import tpu_sc as plsc

E = 8
TOPK = 2
D = 2048
H = 4096
O = 2048
N = 2048

BM = 512
BH = 1024
NH = H // BH
T = N * TOPK
TPAD = T + E * BM
NT = TPAD // BM

BR = 256

NC = 2
NS = 16
NW = NC * NS

GCH = 16
CCH = 16


def _router_body(x_ref, wr_ref, br_ref, vals_ref, idx_ref):
    logits = jnp.dot(x_ref[...], wr_ref[...],
                     preferred_element_type=jnp.float32) + br_ref[...]
    m = jnp.max(logits, axis=1, keepdims=True)
    ex = jnp.exp(logits - m)
    probs = ex / jnp.sum(ex, axis=1, keepdims=True)
    iota = lax.broadcasted_iota(jnp.int32, probs.shape, 1)
    m1 = jnp.max(probs, axis=1, keepdims=True)
    i1 = jnp.min(jnp.where(probs == m1, iota, E), axis=1, keepdims=True)
    probs2 = jnp.where(iota == i1, -1.0, probs)
    m2 = jnp.max(probs2, axis=1, keepdims=True)
    i2 = jnp.min(jnp.where(probs2 == m2, iota, E), axis=1, keepdims=True)
    vals_ref[...] = jnp.concatenate([m1, m2], axis=1)
    idx_ref[...] = jnp.concatenate([i1, i2], axis=1)


def _router(x, Wr, br2):
    return pl.pallas_call(
        _router_body,
        grid=(N // BR,),
        in_specs=[
            pl.BlockSpec((BR, D), lambda i: (i, 0)),
            pl.BlockSpec((D, E), lambda i: (0, 0)),
            pl.BlockSpec((1, E), lambda i: (0, 0)),
        ],
        out_specs=[
            pl.BlockSpec((BR, TOPK), lambda i: (i, 0)),
            pl.BlockSpec((BR, TOPK), lambda i: (i, 0)),
        ],
        out_shape=[
            jax.ShapeDtypeStruct((N, TOPK), jnp.float32),
            jax.ShapeDtypeStruct((N, TOPK), jnp.int32),
        ],
    )(x, Wr, br2)


@functools.cache
def _make_gather_x():
    mesh = plsc.VectorSubcoreMesh(core_axis_name="c", subcore_axis_name="s")

    rpw = TPAD // NW
    nch = rpw // GCH

    @functools.partial(
        pl.kernel,
        mesh=mesh,
        out_type=jax.ShapeDtypeStruct((TPAD, D), jnp.float32),
        scratch_types=[
            pltpu.VMEM((rpw,), jnp.int32),
            pltpu.VMEM((2, GCH, D), jnp.float32),
            pltpu.SemaphoreType.DMA,
            pltpu.SemaphoreType.DMA,
            pltpu.SemaphoreType.DMA,
            pltpu.SemaphoreType.DMA,
        ],
    )
    def _gather_x(x_hbm, idx_hbm, out_hbm, idxs_v, rows_v, gs0, gs1, os0, os1):
        wid = lax.axis_index("s") * NC + lax.axis_index("c")
        base = wid * rpw
        pltpu.sync_copy(idx_hbm.at[pl.ds(base, rpw)], idxs_v)
        gsems = (gs0, gs1)
        osems = (os0, os1)

        def gstart(i, buf):
            return pltpu.async_copy(
                x_hbm.at[idxs_v.at[pl.ds(i * GCH, GCH)]],
                rows_v.at[buf], gsems[buf])

        gcp = gstart(0, 0)
        ocp = [None, None]
        for i in range(nch):
            buf = i % 2
            nxt_cp = None
            if i + 1 < nch:
                nbuf = (i + 1) % 2
                if ocp[nbuf] is not None:
                    ocp[nbuf].wait()
                    ocp[nbuf] = None
                nxt_cp = gstart(i + 1, nbuf)
            gcp.wait()
            ocp[buf] = pltpu.async_copy(
                rows_v.at[buf], out_hbm.at[pl.ds(base + i * GCH, GCH)],
                osems[buf])
            gcp = nxt_cp
        for buf in range(2):
            if ocp[buf] is not None:
                ocp[buf].wait()

    return _gather_x


def _mlp_body(te_ref, tv_ref, xs_ref, w1_ref, b1_ref, w2_ref, b2_ref, g_ref,
              ys_ref, acc_ref):
    t = pl.program_id(0)
    h = pl.program_id(1)

    @pl.when(h == 0)
    def _():
        acc_ref[...] = jnp.zeros_like(acc_ref)

    @pl.when(tv_ref[t] == 1)
    def _():
        hb = jnp.dot(xs_ref[...].astype(jnp.bfloat16),
                     w1_ref[0].astype(jnp.bfloat16),
                     preferred_element_type=jnp.float32)
        hb = jnp.maximum(hb + b1_ref[0], 0.0)
        acc_ref[...] += jnp.dot(hb.astype(jnp.bfloat16),
                                w2_ref[0].astype(jnp.bfloat16),
                                preferred_element_type=jnp.float32)

    @pl.when(h == NH - 1)
    def _():
        ys_ref[...] = (acc_ref[...] + b2_ref[0]) * g_ref[...]


def _mlp(te, tv, xs, W1, b1r, W2, b2r, gates_col):
    grid_spec = pltpu.PrefetchScalarGridSpec(
        num_scalar_prefetch=2,
        grid=(NT, NH),
        in_specs=[
            pl.BlockSpec((BM, D), lambda t, h, te, tv: (t, 0)),
            pl.BlockSpec((1, D, BH), lambda t, h, te, tv: (te[t], 0, h)),
            pl.BlockSpec((1, 1, BH), lambda t, h, te, tv: (te[t], 0, h)),
            pl.BlockSpec((1, BH, O), lambda t, h, te, tv: (te[t], h, 0)),
            pl.BlockSpec((1, 1, O), lambda t, h, te, tv: (te[t], 0, 0)),
            pl.BlockSpec((BM, 1), lambda t, h, te, tv: (t, 0)),
        ],
        out_specs=pl.BlockSpec((BM, O), lambda t, h, te, tv: (t, 0)),
        scratch_shapes=[pltpu.VMEM((BM, O), jnp.float32)],
    )
    return pl.pallas_call(
        _mlp_body,
        grid_spec=grid_spec,
        out_shape=jax.ShapeDtypeStruct((TPAD, O), jnp.float32),
        compiler_params=pltpu.CompilerParams(
            dimension_semantics=("arbitrary", "arbitrary")),
    )(te, tv, xs, W1, b1r, W2, b2r, gates_col)


@functools.cache
def _make_combine():
    mesh = plsc.VectorSubcoreMesh(core_axis_name="c", subcore_axis_name="s")

    @functools.partial(
        pl.kernel,
        mesh=mesh,
        out_type=jax.ShapeDtypeStruct((N, O), jnp.float32),
        scratch_types=[
            pltpu.VMEM((CCH,), jnp.int32),
            pltpu.VMEM((CCH,), jnp.int32),
            pltpu.VMEM((CCH, O), jnp.float32),
            pltpu.VMEM((CCH, O), jnp.float32),
            pltpu.SemaphoreType.DMA,
        ],
    )
    def _combine(ys_hbm, d0_hbm, d1_hbm, out_hbm, i0_v, i1_v, r0_v, r1_v, sem):
        wid = lax.axis_index("s") * NC + lax.axis_index("c")
        rpw = N // NW
        base = wid * rpw

        def chunk(c, carry):
            off = base + c * CCH
            pltpu.sync_copy(d0_hbm.at[pl.ds(off, CCH)], i0_v)
            pltpu.sync_copy(d1_hbm.at[pl.ds(off, CCH)], i1_v)
            cp0 = pltpu.async_copy(ys_hbm.at[i0_v], r0_v, sem)
            cp1 = pltpu.async_copy(ys_hbm.at[i1_v], r1_v, sem)
            cp0.wait()
            cp1.wait()

            def add_lanes(j, carry2):
                s = pl.ds(j * 16, 16)
                for r in range(CCH):
                    r0_v[r, s] = r0_v[r, s] + r1_v[r, s]
                return carry2

            lax.fori_loop(0, O // 16, add_lanes, 0)
            pltpu.sync_copy(r0_v, out_hbm.at[pl.ds(off, CCH)])
            return carry

        lax.fori_loop(0, rpw // CCH, chunk, 0)

    return _combine


def kernel(x, Wr, br, W1, b1, W2, b2):
    top_vals, top_idx = _router(x, Wr, br.reshape(1, E))

    flat_e = top_idx.reshape(-1)
    flat_gate = top_vals.reshape(-1)
    flat_tok = jnp.repeat(jnp.arange(N, dtype=jnp.int32), TOPK)
    onehot = (flat_e[:, None] == jnp.arange(E, dtype=jnp.int32)[None, :])
    csum = jnp.cumsum(onehot.astype(jnp.int32), axis=0)
    counts = csum[-1]
    rank = jnp.take_along_axis(csum, flat_e[:, None], axis=1)[:, 0] - 1
    padded = ((counts + BM - 1) // BM) * BM
    seg_end = jnp.cumsum(padded)
    seg_start = seg_end - padded
    total_active = seg_end[-1]
    dest = seg_start[flat_e] + rank

    spread = jnp.arange(TPAD, dtype=jnp.int32) % N
    tok_pad = spread.at[dest].set(flat_tok)
    gates_pad = jnp.zeros((TPAD,), jnp.float32).at[dest].set(flat_gate)

    tile_start = jnp.arange(NT, dtype=jnp.int32) * BM
    te_raw = jnp.searchsorted(seg_end, tile_start, side="right").astype(jnp.int32)
    valid = tile_start < total_active
    num_active = total_active // BM
    last_e = te_raw[jnp.maximum(num_active - 1, 0)]
    te = jnp.where(valid, jnp.minimum(te_raw, E - 1), last_e)
    tv = valid.astype(jnp.int32)

    xs = _make_gather_x()(x, tok_pad)
    ys = _mlp(te, tv, xs, W1, b1.reshape(E, 1, H), W2, b2.reshape(E, 1, O),
              gates_pad.reshape(TPAD, 1))

    d0 = dest[0::2].astype(jnp.int32)
    d1 = dest[1::2].astype(jnp.int32)
    return _make_combine()(ys, d0, d1)
